# Initial kernel scaffold; baseline (speedup 1.0000x reference)
import functools

import jax
import jax.numpy as jnp
from jax.experimental import pallas as pl
from jax.experimental.pallas import tpu as pltpu
from jax.experimental.pallas import tpu_sc as plsc

N = 10000
E = 320000
D = 128
DE = 16

NC = 2
NS = 16
LANES = 16
K = 80
EPW = E // (NC * NS)
CHUNKS = EPW // K
RPS = N // NS

_HIGHEST = jax.lax.Precision.HIGHEST
_HIGH = jax.lax.Precision.HIGH


def _node_body(x_ref, wn_ref, bn_ref, wm_ref, h_ref, hm_ref):
    h = jnp.maximum(
        jnp.dot(x_ref[...], wn_ref[...], precision=_HIGHEST,
                preferred_element_type=jnp.float32) + bn_ref[...], 0.0)
    h_ref[...] = h
    hm_ref[...] = jnp.dot(h, wm_ref[...], precision=_HIGHEST,
                          preferred_element_type=jnp.float32)


def _node_embed(x, W_node, b_node, W_msg):
    return pl.pallas_call(
        _node_body,
        out_shape=(jax.ShapeDtypeStruct((N, D), jnp.float32),
                   jax.ShapeDtypeStruct((N, D), jnp.float32)),
    )(x, W_node, b_node.reshape(1, D), W_msg)


BE = 4000

def _edge_body(ea_ref, we_ref, be_ref, wm_ref, bm_ref, em_ref):
    e = jnp.maximum(
        jnp.dot(ea_ref[...], we_ref[...], precision=_HIGHEST,
                preferred_element_type=jnp.float32) + be_ref[...], 0.0)
    em_ref[...] = jnp.dot(e, wm_ref[...], precision=_HIGH,
                          preferred_element_type=jnp.float32) + bm_ref[...]


def _edge_embed(edge_attr, W_edge, b_edge, W_msg, b_msg):
    return pl.pallas_call(
        _edge_body,
        grid=(E // BE,),
        in_specs=[
            pl.BlockSpec((BE, DE), lambda i: (i, 0)),
            pl.BlockSpec((DE, D), lambda i: (0, 0)),
            pl.BlockSpec((1, D), lambda i: (0, 0)),
            pl.BlockSpec((D, D), lambda i: (0, 0)),
            pl.BlockSpec((1, D), lambda i: (0, 0)),
        ],
        out_specs=pl.BlockSpec((BE, D), lambda i: (i, 0)),
        out_shape=jax.ShapeDtypeStruct((E, D), jnp.float32),
    )(edge_attr, W_edge, b_edge.reshape(1, D), W_msg, b_msg.reshape(1, D))


def _sc_body(hm_hbm, em_hbm, src_hbm, dst_hbm, acc_hbm, cnt_hbm,
             src_v, dst_v, em_v, g_v, ones_v, zrow_v, zc_v,
             acc_sh, cnt_sh, sem):
    c = jax.lax.axis_index("c")
    s = jax.lax.axis_index("s")
    wid = c * NS + s

    @pl.loop(0, 125)
    def _(r):
        @pl.loop(0, D, step=LANES)
        def _(j):
            zrow_v[r, pl.ds(j, LANES)] = jnp.zeros((LANES,), jnp.float32)

    @pl.loop(0, RPS)
    def _(r):
        zc_v[r, pl.ds(0, LANES)] = jnp.zeros((LANES,), jnp.float32)

    @pl.loop(0, K)
    def _(r):
        ones_v[r, pl.ds(0, LANES)] = jnp.full((LANES,), 1.0, jnp.float32)

    @pl.loop(0, RPS // 125)
    def _(q):
        pltpu.sync_copy(zrow_v, acc_sh.at[pl.ds(s * RPS + q * 125, 125)])
    pltpu.sync_copy(zc_v, cnt_sh.at[pl.ds(s * RPS, RPS)])
    plsc.subcore_barrier()

    @pl.loop(0, CHUNKS)
    def _(t):
        base = wid * EPW + t * K
        pltpu.sync_copy(src_hbm.at[pl.ds(base, K)], src_v)
        pltpu.sync_copy(dst_hbm.at[pl.ds(base, K)], dst_v)
        pltpu.sync_copy(em_hbm.at[pl.ds(base, K)], em_v)
        pltpu.async_copy(hm_hbm.at[src_v], g_v, sem).wait()

        @pl.loop(0, K)
        def _(r):
            @pl.loop(0, D, step=LANES)
            def _(j):
                sl = pl.ds(j, LANES)
                g_v[r, sl] = jnp.maximum(g_v[r, sl] + em_v[r, sl], 0.0)

        pltpu.sync_copy(g_v, acc_sh.at[dst_v], add=True)
        pltpu.sync_copy(ones_v, cnt_sh.at[dst_v], add=True)

    plsc.subcore_barrier()

    pltpu.sync_copy(acc_sh.at[pl.ds(s * RPS, RPS)],
                    acc_hbm.at[c, pl.ds(s * RPS, RPS)])
    pltpu.sync_copy(cnt_sh.at[pl.ds(s * RPS, RPS)],
                    cnt_hbm.at[c, pl.ds(s * RPS, RPS)])


def _sc_aggregate(hm, em, src, dst):
    mesh = plsc.VectorSubcoreMesh(core_axis_name="c", subcore_axis_name="s")
    kern = pl.kernel(
        _sc_body,
        out_type=(jax.ShapeDtypeStruct((NC, N, D), jnp.float32),
                  jax.ShapeDtypeStruct((NC, N, LANES), jnp.float32)),
        mesh=mesh,
        scratch_types=[
            pltpu.VMEM((K,), jnp.int32),
            pltpu.VMEM((K,), jnp.int32),
            pltpu.VMEM((K, D), jnp.float32),
            pltpu.VMEM((K, D), jnp.float32),
            pltpu.VMEM((K, LANES), jnp.float32),
            pltpu.VMEM((125, D), jnp.float32),
            pltpu.VMEM((RPS, LANES), jnp.float32),
            pltpu.VMEM_SHARED((N, D), jnp.float32),
            pltpu.VMEM_SHARED((N, LANES), jnp.float32),
            pltpu.SemaphoreType.DMA,
        ],
    )
    return kern(hm, em, src, dst)


def _upd_body(h_ref, acc_ref, cnt_ref, w1_ref, w2_ref, w3_ref, bu_ref, o_ref):
    agg = acc_ref[0] + acc_ref[1]
    cnt = cnt_ref[0, :, 0:1] + cnt_ref[1, :, 0:1]
    mean = agg / jnp.maximum(cnt, 1.0)
    o = (jnp.dot(h_ref[...], w1_ref[...], precision=_HIGHEST,
                 preferred_element_type=jnp.float32)
         + jnp.dot(agg, w2_ref[...], precision=_HIGHEST,
                   preferred_element_type=jnp.float32)
         + jnp.dot(mean, w3_ref[...], precision=_HIGHEST,
                   preferred_element_type=jnp.float32)
         + bu_ref[...])
    o_ref[...] = jnp.maximum(o, 0.0)


def _node_update(h, acc, cnt, W_upd, b_upd):
    return pl.pallas_call(
        _upd_body,
        out_shape=jax.ShapeDtypeStruct((N, D), jnp.float32),
    )(h, acc, cnt, W_upd[0:D], W_upd[D:2 * D], W_upd[2 * D:3 * D],
      b_upd.reshape(1, D))


def kernel(x, edge_index, edge_attr, W_node, b_node, W_edge, b_edge,
           W_msg, b_msg, W_upd, b_upd):
    src = edge_index[0].astype(jnp.int32)
    dst = edge_index[1].astype(jnp.int32)
    h, hm = _node_embed(x, W_node, b_node, W_msg)
    em = _edge_embed(edge_attr, W_edge, b_edge, W_msg, b_msg)
    acc, cnt = _sc_aggregate(hm, em, src, dst)
    return _node_update(h, acc, cnt, W_upd, b_upd)


# speedup vs baseline: 1.8654x; 1.8654x over previous
import dataclasses

import jax
import jax.numpy as jnp
from jax.experimental import pallas as pl
from jax.experimental.pallas import tpu as pltpu
from jax.experimental.pallas import tpu_sc as plsc

N = 10000
E = 320000
D = 128
DE = 16

NC = 2
NS = 16
LANES = 16
K = 80
EPW = E // (NC * NS)
CHUNKS = EPW // K
NP = 10240
RPS = NP // NS
NQ = 1280
QPS = NQ // NS

_HIGHEST = jax.lax.Precision.HIGHEST


def _node_body(x_ref, wn_ref, bn_ref, wm_ref, h_ref, hm_ref):
    h = jnp.maximum(
        jnp.dot(x_ref[...], wn_ref[...], precision=_HIGHEST,
                preferred_element_type=jnp.float32) + bn_ref[...], 0.0)
    h_ref[...] = h
    hm_ref[...] = jnp.dot(h, wm_ref[...], precision=_HIGHEST,
                          preferred_element_type=jnp.float32)


def _node_embed(x, W_node, b_node, W_msg):
    return pl.pallas_call(
        _node_body,
        out_shape=(jax.ShapeDtypeStruct((N, D), jnp.float32),
                   jax.ShapeDtypeStruct((N, D), jnp.float32)),
    )(x, W_node, b_node.reshape(1, D), W_msg)


BE = 4000

def _edge_body(ea_ref, we_ref, be_ref, wm_ref, bm_ref, em_ref):
    e = jnp.maximum(
        jnp.dot(ea_ref[...], we_ref[...], precision=_HIGHEST,
                preferred_element_type=jnp.float32) + be_ref[...], 0.0)
    em_ref[...] = jnp.dot(e, wm_ref[...], precision=_HIGHEST,
                          preferred_element_type=jnp.float32) + bm_ref[...]


def _edge_embed(edge_attr, W_edge, b_edge, W_msg, b_msg):
    return pl.pallas_call(
        _edge_body,
        grid=(E // BE,),
        in_specs=[
            pl.BlockSpec((BE, DE), lambda i: (i, 0)),
            pl.BlockSpec((DE, D), lambda i: (0, 0)),
            pl.BlockSpec((1, D), lambda i: (0, 0)),
            pl.BlockSpec((D, D), lambda i: (0, 0)),
            pl.BlockSpec((1, D), lambda i: (0, 0)),
        ],
        out_specs=pl.BlockSpec((BE, D), lambda i: (i, 0)),
        out_shape=jax.ShapeDtypeStruct((E, D), jnp.float32),
    )(edge_attr, W_edge, b_edge.reshape(1, D), W_msg, b_msg.reshape(1, D))


def _sc_body(hm_hbm, em_hbm, src_hbm, dst_hbm, acc_hbm, aux_hbm,
             src_v, dst_v, dstq_v, em_v, g_v, ones_v,
             acc_sh, aux_sh, sem):
    c = jax.lax.axis_index("c")
    s = jax.lax.axis_index("s")
    wid = c * NS + s

    @pl.loop(0, K)
    def _(r):
        @pl.loop(0, D, step=LANES)
        def _(j):
            g_v[r, pl.ds(j, LANES)] = jnp.zeros((LANES,), jnp.float32)
            ones_v[r, pl.ds(j, LANES)] = jnp.zeros((LANES,), jnp.float32)

    @pl.loop(0, RPS // K)
    def _(q):
        pltpu.sync_copy(g_v, acc_sh.at[pl.ds(s * RPS + q * K, K)])
    pltpu.sync_copy(g_v, aux_sh.at[pl.ds(s * QPS, QPS)])
    plsc.subcore_barrier()

    @pl.loop(0, CHUNKS)
    def _(t):
        base = wid * EPW + t * K
        pltpu.sync_copy(src_hbm.at[pl.ds(base, K)], src_v)
        pltpu.sync_copy(dst_hbm.at[pl.ds(base, K)], dst_v)
        pltpu.sync_copy(em_hbm.at[pl.ds(base, K)], em_v)
        pltpu.async_copy(hm_hbm.at[src_v], g_v, sem).wait()

        @pl.loop(0, K, step=LANES)
        def _(r16):
            sl = pl.ds(r16, LANES)
            d16 = dst_v[sl]
            dstq_v[sl] = jax.lax.shift_right_logical(d16, 3)
            rows = jax.lax.iota(jnp.int32, LANES) + r16
            cols = (d16 & 7) * LANES
            plsc.store_scatter(ones_v, [rows, cols],
                               jnp.full((LANES,), 1.0, jnp.float32))

        @pl.loop(0, K)
        def _(r):
            @pl.loop(0, D, step=LANES)
            def _(j):
                sl = pl.ds(j, LANES)
                g_v[r, sl] = jnp.maximum(g_v[r, sl] + em_v[r, sl], 0.0)

        pltpu.sync_copy(g_v, acc_sh.at[dst_v], add=True)
        pltpu.sync_copy(ones_v, aux_sh.at[dstq_v], add=True)

        @pl.loop(0, K, step=LANES)
        def _(r16):
            sl = pl.ds(r16, LANES)
            rows = jax.lax.iota(jnp.int32, LANES) + r16
            cols = (dst_v[sl] & 7) * LANES
            plsc.store_scatter(ones_v, [rows, cols],
                               jnp.zeros((LANES,), jnp.float32))

    plsc.subcore_barrier()

    pltpu.sync_copy(acc_sh.at[pl.ds(s * RPS, RPS)],
                    acc_hbm.at[c, pl.ds(s * RPS, RPS)])
    pltpu.sync_copy(aux_sh.at[pl.ds(s * QPS, QPS)],
                    aux_hbm.at[c, pl.ds(s * QPS, QPS)])


def _sc_aggregate(hm, em, src, dst):
    mesh = plsc.VectorSubcoreMesh(core_axis_name="c", subcore_axis_name="s")
    cp = pltpu.CompilerParams()
    if "needs_layout_passes" in pltpu.CompilerParams.__dataclass_fields__:
        cp = dataclasses.replace(cp, needs_layout_passes=False)
    kern = pl.kernel(
        _sc_body,
        compiler_params=cp,
        out_type=(jax.ShapeDtypeStruct((NC, NP, D), jnp.float32),
                  jax.ShapeDtypeStruct((NC, NQ, D), jnp.float32)),
        mesh=mesh,
        scratch_types=[
            pltpu.VMEM((K,), jnp.int32),
            pltpu.VMEM((K,), jnp.int32),
            pltpu.VMEM((K,), jnp.int32),
            pltpu.VMEM((K, D), jnp.float32),
            pltpu.VMEM((K, D), jnp.float32),
            pltpu.VMEM((K, D), jnp.float32),
            pltpu.VMEM_SHARED((NP, D), jnp.float32),
            pltpu.VMEM_SHARED((NQ, D), jnp.float32),
            pltpu.SemaphoreType.DMA,
        ],
    )
    return kern(hm, em, src, dst)


BN = 2000

def _upd_body(h_ref, acc_ref, cnt_ref, w1_ref, w2_ref, w3_ref, bu_ref, o_ref):
    agg = acc_ref[0] + acc_ref[1]
    cnt = cnt_ref[...]
    mean = agg / jnp.maximum(cnt, 1.0)
    o = (jnp.dot(h_ref[...], w1_ref[...], precision=_HIGHEST,
                 preferred_element_type=jnp.float32)
         + jnp.dot(agg, w2_ref[...], precision=_HIGHEST,
                   preferred_element_type=jnp.float32)
         + jnp.dot(mean, w3_ref[...], precision=_HIGHEST,
                   preferred_element_type=jnp.float32)
         + bu_ref[...])
    o_ref[...] = jnp.maximum(o, 0.0)


def _node_update(h, acc, cnt, W_upd, b_upd):
    return pl.pallas_call(
        _upd_body,
        grid=(N // BN,),
        in_specs=[
            pl.BlockSpec((BN, D), lambda i: (i, 0)),
            pl.BlockSpec((2, BN, D), lambda i: (0, i, 0)),
            pl.BlockSpec((BN, 1), lambda i: (i, 0)),
            pl.BlockSpec((D, D), lambda i: (0, 0)),
            pl.BlockSpec((D, D), lambda i: (0, 0)),
            pl.BlockSpec((D, D), lambda i: (0, 0)),
            pl.BlockSpec((1, D), lambda i: (0, 0)),
        ],
        out_specs=pl.BlockSpec((BN, D), lambda i: (i, 0)),
        out_shape=jax.ShapeDtypeStruct((N, D), jnp.float32),
    )(h, acc, cnt, W_upd[0:D], W_upd[D:2 * D], W_upd[2 * D:3 * D],
      b_upd.reshape(1, D))


def kernel(x, edge_index, edge_attr, W_node, b_node, W_edge, b_edge,
           W_msg, b_msg, W_upd, b_upd):
    src = edge_index[0].astype(jnp.int32)
    dst = edge_index[1].astype(jnp.int32)
    h, hm = _node_embed(x, W_node, b_node, W_msg)
    em = _edge_embed(edge_attr, W_edge, b_edge, W_msg, b_msg)
    acc, aux = _sc_aggregate(hm, em, src, dst)
    auxs = aux[0] + aux[1]
    cnt = auxs.reshape(NQ, 8, LANES)[:, :, 0].reshape(NQ * 8, 1)[:N]
    return _node_update(h, acc, cnt, W_upd, b_upd)


# speedup vs baseline: 2.6126x; 1.4006x over previous
import dataclasses

import jax
import jax.numpy as jnp
from jax.experimental import pallas as pl
from jax.experimental.pallas import tpu as pltpu
from jax.experimental.pallas import tpu_sc as plsc

N = 10000
E = 320000
D = 128
DE = 16

NC = 2
NS = 16
LANES = 16
K = 80
EPW = E // (NC * NS)
CHUNKS = EPW // K
NP = 10240
RPS = NP // NS
NQ = 1280
QPS = NQ // NS

_HIGHEST = jax.lax.Precision.HIGHEST


def _node_body(x_ref, wn_ref, bn_ref, wm_ref, h_ref, hm_ref):
    h = jnp.maximum(
        jnp.dot(x_ref[...], wn_ref[...], precision=_HIGHEST,
                preferred_element_type=jnp.float32) + bn_ref[...], 0.0)
    h_ref[...] = h
    hm_ref[...] = jnp.dot(h, wm_ref[...], precision=_HIGHEST,
                          preferred_element_type=jnp.float32)


def _node_embed(x, W_node, b_node, W_msg):
    return pl.pallas_call(
        _node_body,
        out_shape=(jax.ShapeDtypeStruct((N, D), jnp.float32),
                   jax.ShapeDtypeStruct((N, D), jnp.float32)),
    )(x, W_node, b_node.reshape(1, D), W_msg)


BE2 = 2000

def _edge_body(ea_ref, w1_ref, b1_ref, w2_ref, b2_ref, em_ref):
    ea = ea_ref[...].astype(jnp.bfloat16)
    z = jnp.dot(ea, w1_ref[...], preferred_element_type=jnp.float32)
    e = jnp.maximum(z + b1_ref[...], 0.0).astype(jnp.bfloat16)
    em_ref[...] = jnp.dot(e, w2_ref[...],
                          preferred_element_type=jnp.float32) + b2_ref[...]


def _edge_embed(edge_attr, W_edge, b_edge, W_msg, b_msg):
    z2 = jnp.zeros((DE, D), jnp.float32)
    w1 = jnp.block([[W_edge, z2], [z2, W_edge]]).astype(jnp.bfloat16)
    zd = jnp.zeros((D, D), jnp.float32)
    w2 = jnp.block([[W_msg, zd], [zd, W_msg]]).astype(jnp.bfloat16)
    b1 = jnp.concatenate([b_edge, b_edge]).reshape(1, 2 * D)
    b2 = jnp.concatenate([b_msg, b_msg]).reshape(1, 2 * D)
    ea2 = edge_attr.reshape(E // 2, 2 * DE)
    em2 = pl.pallas_call(
        _edge_body,
        grid=(E // 2 // BE2,),
        in_specs=[
            pl.BlockSpec((BE2, 2 * DE), lambda i: (i, 0)),
            pl.BlockSpec((2 * DE, 2 * D), lambda i: (0, 0)),
            pl.BlockSpec((1, 2 * D), lambda i: (0, 0)),
            pl.BlockSpec((2 * D, 2 * D), lambda i: (0, 0)),
            pl.BlockSpec((1, 2 * D), lambda i: (0, 0)),
        ],
        out_specs=pl.BlockSpec((BE2, 2 * D), lambda i: (i, 0)),
        out_shape=jax.ShapeDtypeStruct((E // 2, 2 * D), jnp.float32),
    )(ea2, w1, b1, w2, b2)
    return em2.reshape(E, D)


def _sc_body(hm_hbm, em_hbm, src_hbm, dst_hbm, acc_hbm, aux_hbm,
             src_v, dst_v, dstq_v, em_v, g_v, ones_v,
             acc_sh, aux_sh, sem, sem2, sem3, sem4):
    c = jax.lax.axis_index("c")
    s = jax.lax.axis_index("s")
    wid = c * NS + s

    @pl.loop(0, K)
    def _(r):
        @pl.loop(0, D, step=LANES)
        def _(j):
            g_v[r, pl.ds(j, LANES)] = jnp.zeros((LANES,), jnp.float32)
            ones_v[r, pl.ds(j, LANES)] = jnp.zeros((LANES,), jnp.float32)

    @pl.loop(0, RPS // K)
    def _(q):
        pltpu.sync_copy(g_v, acc_sh.at[pl.ds(s * RPS + q * K, K)])
    pltpu.sync_copy(g_v, aux_sh.at[pl.ds(s * QPS, QPS)])
    plsc.subcore_barrier()

    @pl.loop(0, CHUNKS)
    def _(t):
        base = wid * EPW + t * K
        cp_src = pltpu.async_copy(src_hbm.at[pl.ds(base, K)], src_v, sem2)
        cp_dst = pltpu.async_copy(dst_hbm.at[pl.ds(base, K)], dst_v, sem3)
        cp_em = pltpu.async_copy(em_hbm.at[pl.ds(base, K)], em_v, sem4)
        cp_src.wait()
        cp_gather = pltpu.async_copy(hm_hbm.at[src_v], g_v, sem)
        cp_dst.wait()

        @pl.loop(0, K, step=LANES)
        def _(r16):
            sl = pl.ds(r16, LANES)
            d16 = dst_v[sl]
            dstq_v[sl] = jax.lax.shift_right_logical(d16, 3)
            rows = jax.lax.iota(jnp.int32, LANES) + r16
            cols = (d16 & 7) * LANES
            plsc.store_scatter(ones_v, [rows, cols],
                               jnp.full((LANES,), 1.0, jnp.float32))

        cp_em.wait()
        cp_gather.wait()

        @pl.loop(0, K)
        def _(r):
            for j in range(0, D, LANES):
                sl = pl.ds(j, LANES)
                g_v[r, sl] = jnp.maximum(g_v[r, sl] + em_v[r, sl], 0.0)

        cp_acc = pltpu.async_copy(g_v, acc_sh.at[dst_v], sem2, add=True)
        pltpu.sync_copy(ones_v, aux_sh.at[dstq_v], add=True)
        cp_acc.wait()

        @pl.loop(0, K, step=LANES)
        def _(r16):
            sl = pl.ds(r16, LANES)
            rows = jax.lax.iota(jnp.int32, LANES) + r16
            cols = (dst_v[sl] & 7) * LANES
            plsc.store_scatter(ones_v, [rows, cols],
                               jnp.zeros((LANES,), jnp.float32))

    plsc.subcore_barrier()

    pltpu.sync_copy(acc_sh.at[pl.ds(s * RPS, RPS)],
                    acc_hbm.at[c, pl.ds(s * RPS, RPS)])
    pltpu.sync_copy(aux_sh.at[pl.ds(s * QPS, QPS)],
                    aux_hbm.at[c, pl.ds(s * QPS, QPS)])


def _sc_aggregate(hm, em, src, dst):
    mesh = plsc.VectorSubcoreMesh(core_axis_name="c", subcore_axis_name="s")
    cp = pltpu.CompilerParams()
    if "needs_layout_passes" in pltpu.CompilerParams.__dataclass_fields__:
        cp = dataclasses.replace(cp, needs_layout_passes=False)
    kern = pl.kernel(
        _sc_body,
        compiler_params=cp,
        out_type=(jax.ShapeDtypeStruct((NC, NP, D), jnp.float32),
                  jax.ShapeDtypeStruct((NC, NQ, D), jnp.float32)),
        mesh=mesh,
        scratch_types=[
            pltpu.VMEM((K,), jnp.int32),
            pltpu.VMEM((K,), jnp.int32),
            pltpu.VMEM((K,), jnp.int32),
            pltpu.VMEM((K, D), jnp.float32),
            pltpu.VMEM((K, D), jnp.float32),
            pltpu.VMEM((K, D), jnp.float32),
            pltpu.VMEM_SHARED((NP, D), jnp.float32),
            pltpu.VMEM_SHARED((NQ, D), jnp.float32),
            pltpu.SemaphoreType.DMA,
            pltpu.SemaphoreType.DMA,
            pltpu.SemaphoreType.DMA,
            pltpu.SemaphoreType.DMA,
        ],
    )
    return kern(hm, em, src, dst)


BN = 2000

def _upd_body(h_ref, acc_ref, cnt_ref, w1_ref, w2_ref, w3_ref, bu_ref, o_ref):
    agg = acc_ref[0] + acc_ref[1]
    cnt = cnt_ref[...]
    mean = agg / jnp.maximum(cnt, 1.0)
    o = (jnp.dot(h_ref[...], w1_ref[...], precision=_HIGHEST,
                 preferred_element_type=jnp.float32)
         + jnp.dot(agg, w2_ref[...], precision=_HIGHEST,
                   preferred_element_type=jnp.float32)
         + jnp.dot(mean, w3_ref[...], precision=_HIGHEST,
                   preferred_element_type=jnp.float32)
         + bu_ref[...])
    o_ref[...] = jnp.maximum(o, 0.0)


def _node_update(h, acc, cnt, W_upd, b_upd):
    return pl.pallas_call(
        _upd_body,
        grid=(N // BN,),
        in_specs=[
            pl.BlockSpec((BN, D), lambda i: (i, 0)),
            pl.BlockSpec((2, BN, D), lambda i: (0, i, 0)),
            pl.BlockSpec((BN, 1), lambda i: (i, 0)),
            pl.BlockSpec((D, D), lambda i: (0, 0)),
            pl.BlockSpec((D, D), lambda i: (0, 0)),
            pl.BlockSpec((D, D), lambda i: (0, 0)),
            pl.BlockSpec((1, D), lambda i: (0, 0)),
        ],
        out_specs=pl.BlockSpec((BN, D), lambda i: (i, 0)),
        out_shape=jax.ShapeDtypeStruct((N, D), jnp.float32),
    )(h, acc, cnt, W_upd[0:D], W_upd[D:2 * D], W_upd[2 * D:3 * D],
      b_upd.reshape(1, D))


def kernel(x, edge_index, edge_attr, W_node, b_node, W_edge, b_edge,
           W_msg, b_msg, W_upd, b_upd):
    src = edge_index[0].astype(jnp.int32)
    dst = edge_index[1].astype(jnp.int32)
    h, hm = _node_embed(x, W_node, b_node, W_msg)
    em = _edge_embed(edge_attr, W_edge, b_edge, W_msg, b_msg)
    acc, aux = _sc_aggregate(hm, em, src, dst)
    auxs = aux[0] + aux[1]
    cnt = auxs.reshape(NQ, 8, LANES)[:, :, 0].reshape(NQ * 8, 1)[:N]
    return _node_update(h, acc, cnt, W_upd, b_upd)
